# Initial kernel scaffold; baseline (speedup 1.0000x reference)
import jax
import jax.numpy as jnp
from jax.experimental import pallas as pl


def kernel(x, table):
    raise NotImplementedError("write your pallas kernel here")


# speedup vs baseline: 1.2331x; 1.2331x over previous
import functools

import jax
import jax.numpy as jnp
from jax import lax
from jax.experimental import pallas as pl
from jax.experimental.pallas import tpu as pltpu
from jax.experimental.pallas import tpu_sc as plsc

BATCH = 4096
N_FIELDS = 26
EMB_DIM = 128
TOTAL = BATCH * N_FIELDS

NUM_CORES = 2
NUM_SUBCORES = 16
NUM_WORKERS = NUM_CORES * NUM_SUBCORES
PER_W = TOTAL // NUM_WORKERS
CHUNK = 128
NCHUNK = PER_W // CHUNK


def _gather_body(x_hbm, table_hbm, out_hbm, idx_v, buf_v, gsem):
    wid = lax.axis_index("s") * NUM_CORES + lax.axis_index("c")
    base = wid * PER_W
    pltpu.sync_copy(x_hbm.at[wid], idx_v)
    for j in range(NCHUNK):
        pltpu.async_copy(table_hbm.at[idx_v.at[j]], buf_v, gsem).wait()
        pltpu.sync_copy(buf_v, out_hbm.at[pl.ds(base + j * CHUNK, CHUNK)])


@jax.jit
def _sc_gather(x_flat, table):
    mesh = plsc.VectorSubcoreMesh(core_axis_name="c", subcore_axis_name="s")
    k = functools.partial(
        pl.kernel,
        out_type=jax.ShapeDtypeStruct((TOTAL, EMB_DIM), jnp.float32),
        mesh=mesh,
        scratch_types=[
            pltpu.VMEM((NCHUNK, CHUNK), jnp.int32),
            pltpu.VMEM((CHUNK, EMB_DIM), jnp.float32),
            pltpu.SemaphoreType.DMA,
        ],
    )(_gather_body)
    return k(x_flat, table)


def kernel(x, table):
    x_flat = x.reshape(NUM_WORKERS, NCHUNK, CHUNK)
    out = _sc_gather(x_flat, table)
    return out.reshape(BATCH, N_FIELDS, 1, 8, 16)


# speedup vs baseline: 1.2748x; 1.0338x over previous
import functools

import jax
import jax.numpy as jnp
from jax import lax
from jax.experimental import pallas as pl
from jax.experimental.pallas import tpu as pltpu
from jax.experimental.pallas import tpu_sc as plsc

BATCH = 4096
N_FIELDS = 26
EMB_DIM = 128
TOTAL = BATCH * N_FIELDS

NUM_CORES = 2
NUM_SUBCORES = 16
NUM_WORKERS = NUM_CORES * NUM_SUBCORES
PER_W = TOTAL // NUM_WORKERS
CHUNK = 128
NCHUNK = PER_W // CHUNK


NBUF = 6


def _gather_body(x_hbm, table_hbm, out_hbm, idx_v, bufs_v, *sems):
    gsems, ssems = sems[:NBUF], sems[NBUF:]
    wid = lax.axis_index("s") * NUM_CORES + lax.axis_index("c")
    base = wid * PER_W
    pltpu.sync_copy(x_hbm.at[wid], idx_v)

    def gather(j):
        b = j % NBUF
        return pltpu.async_copy(table_hbm.at[idx_v.at[j]], bufs_v.at[b], gsems[b])

    def store(j):
        b = j % NBUF
        return pltpu.async_copy(
            bufs_v.at[b], out_hbm.at[pl.ds(base + j * CHUNK, CHUNK)], ssems[b]
        )

    gathers = [None] * NCHUNK
    stores = [None] * NCHUNK
    for j in range(NBUF - 1):
        gathers[j] = gather(j)
    for j in range(NCHUNK):
        nj = j + NBUF - 1
        if nj < NCHUNK:
            if j >= 1:
                stores[j - 1].wait()
            gathers[nj] = gather(nj)
        gathers[j].wait()
        stores[j] = store(j)
    for j in range(max(0, NCHUNK - NBUF), NCHUNK):
        stores[j].wait()


@jax.jit
def _sc_gather(x_flat, table):
    mesh = plsc.VectorSubcoreMesh(core_axis_name="c", subcore_axis_name="s")
    k = functools.partial(
        pl.kernel,
        out_type=jax.ShapeDtypeStruct((TOTAL, EMB_DIM), jnp.float32),
        mesh=mesh,
        scratch_types=[
            pltpu.VMEM((NCHUNK, CHUNK), jnp.int32),
            pltpu.VMEM((NBUF, CHUNK, EMB_DIM), jnp.float32),
        ]
        + [pltpu.SemaphoreType.DMA] * (2 * NBUF),
    )(_gather_body)
    return k(x_flat, table)


def kernel(x, table):
    x_flat = x.reshape(NUM_WORKERS, NCHUNK, CHUNK)
    out = _sc_gather(x_flat, table)
    return out.reshape(BATCH, N_FIELDS, 1, 8, 16)


# speedup vs baseline: 1.2769x; 1.0017x over previous
import functools

import jax
import jax.numpy as jnp
from jax import lax
from jax.experimental import pallas as pl
from jax.experimental.pallas import tpu as pltpu
from jax.experimental.pallas import tpu_sc as plsc

BATCH = 4096
N_FIELDS = 26
EMB_DIM = 128
TOTAL = BATCH * N_FIELDS

NUM_CORES = 2
NUM_SUBCORES = 16
NUM_WORKERS = NUM_CORES * NUM_SUBCORES
ROWS_W = BATCH // NUM_WORKERS
PER_W = ROWS_W * N_FIELDS
CHUNK = 128
NCHUNK = PER_W // CHUNK
NBUF = 6


def _gather_body(x_hbm, table_hbm, out_hbm, idx2d_v, fidx_v, bufs_v, *sems):
    gsems, ssems = sems[:NBUF], sems[NBUF:]
    wid = lax.axis_index("s") * NUM_CORES + lax.axis_index("c")
    base = wid * PER_W
    pltpu.sync_copy(x_hbm.at[pl.ds(wid * ROWS_W, ROWS_W)], idx2d_v)

    @pl.loop(0, ROWS_W)
    def _flatten(r):
        fidx_v[pl.ds(r * N_FIELDS, 16)] = idx2d_v[r, pl.ds(0, 16)]
        fidx_v[pl.ds(r * N_FIELDS + 10, 16)] = idx2d_v[r, pl.ds(10, 16)]

    def gather(j):
        b = j % NBUF
        return pltpu.async_copy(
            table_hbm.at[fidx_v.at[pl.ds(j * CHUNK, CHUNK)]], bufs_v.at[b], gsems[b]
        )

    def store(j):
        b = j % NBUF
        return pltpu.async_copy(
            bufs_v.at[b], out_hbm.at[pl.ds(base + j * CHUNK, CHUNK)], ssems[b]
        )

    gathers = [None] * NCHUNK
    stores = [None] * NCHUNK
    for j in range(NBUF - 1):
        gathers[j] = gather(j)
    for j in range(NCHUNK):
        nj = j + NBUF - 1
        if nj < NCHUNK:
            if j >= 1:
                stores[j - 1].wait()
            gathers[nj] = gather(nj)
        gathers[j].wait()
        stores[j] = store(j)
    for j in range(max(0, NCHUNK - NBUF), NCHUNK):
        stores[j].wait()


@jax.jit
def _sc_gather(x, table):
    mesh = plsc.VectorSubcoreMesh(core_axis_name="c", subcore_axis_name="s")
    k = functools.partial(
        pl.kernel,
        out_type=jax.ShapeDtypeStruct((TOTAL, EMB_DIM), jnp.float32),
        mesh=mesh,
        scratch_types=[
            pltpu.VMEM((ROWS_W, N_FIELDS), jnp.int32),
            pltpu.VMEM((PER_W,), jnp.int32),
            pltpu.VMEM((NBUF, CHUNK, EMB_DIM), jnp.float32),
        ]
        + [pltpu.SemaphoreType.DMA] * (2 * NBUF),
    )(_gather_body)
    return k(x, table)


def kernel(x, table):
    out = _sc_gather(x, table)
    return out.reshape(BATCH, N_FIELDS, 1, 8, 16)


# speedup vs baseline: 2.2883x; 1.7920x over previous
import functools

import jax
import jax.numpy as jnp
from jax import lax
from jax.experimental import pallas as pl
from jax.experimental.pallas import tpu as pltpu
from jax.experimental.pallas import tpu_sc as plsc

BATCH = 4096
N_FIELDS = 26
EMB_DIM = 128

NUM_CORES = 2
NUM_SUBCORES = 16
NUM_WORKERS = NUM_CORES * NUM_SUBCORES
BLK = BATCH // NUM_WORKERS


def _gather_t_body(xt_hbm, table_hbm, out_hbm, idx_v, gbufs, tbufs, *sems):
    gsems, ssems = sems[:2], sems[2:]
    wid = lax.axis_index("s") * NUM_CORES + lax.axis_index("c")
    pltpu.sync_copy(xt_hbm.at[:, pl.ds(wid * BLK, BLK)], idx_v)

    def gather(f):
        return pltpu.async_copy(
            table_hbm.at[idx_v.at[f]], gbufs.at[f % 2], gsems[f % 2]
        )

    def store(f):
        return pltpu.async_copy(tbufs.at[f % 2], out_hbm.at[f, :, wid], ssems[f % 2])

    iota = lax.iota(jnp.int32, 16)
    bvecs = [iota + (b0 * 16) for b0 in range(8)]

    def transpose(f):
        g = gbufs.at[f % 2]
        t = tbufs.at[f % 2]

        @pl.loop(0, EMB_DIM)
        def _row(d):
            dsp = jnp.full((16,), d, jnp.int32)
            dt = lax.shift_right_logical(d, 3)
            ds = lax.bitwise_and(d, 7)
            for b0 in range(8):
                t[dt, ds, pl.ds(b0 * 16, 16)] = plsc.load_gather(g, [bvecs[b0], dsp])

    gathers = [None] * N_FIELDS
    stores = [None] * N_FIELDS
    gathers[0] = gather(0)
    gathers[1] = gather(1)
    for f in range(N_FIELDS):
        gathers[f].wait()
        if f >= 2:
            stores[f - 2].wait()
        transpose(f)
        stores[f] = store(f)
        if f + 2 < N_FIELDS:
            gathers[f + 2] = gather(f + 2)
    stores[N_FIELDS - 2].wait()
    stores[N_FIELDS - 1].wait()


@jax.jit
def _sc_gather_t(xt, table):
    mesh = plsc.VectorSubcoreMesh(core_axis_name="c", subcore_axis_name="s")
    k = functools.partial(
        pl.kernel,
        out_type=jax.ShapeDtypeStruct(
            (N_FIELDS, EMB_DIM // 8, NUM_WORKERS, 8, BLK), jnp.float32
        ),
        mesh=mesh,
        scratch_types=[
            pltpu.VMEM((N_FIELDS, BLK), jnp.int32),
            pltpu.VMEM((2, BLK, EMB_DIM), jnp.float32),
            pltpu.VMEM((2, EMB_DIM // 8, 8, BLK), jnp.float32),
        ]
        + [pltpu.SemaphoreType.DMA] * 4,
        compiler_params=pltpu.CompilerParams(
            use_tc_tiling_on_sc=False, needs_layout_passes=False
        ),
    )(_gather_t_body)
    return k(xt, table)


def kernel(x, table):
    raw = _sc_gather_t(x.T, table)
    return raw.transpose(2, 4, 0, 1, 3).reshape(BATCH, N_FIELDS, 1, 8, 16)


# speedup vs baseline: 2.7893x; 1.2189x over previous
import functools

import jax
import jax.numpy as jnp
from jax import lax
from jax.experimental import pallas as pl
from jax.experimental.pallas import tpu as pltpu
from jax.experimental.pallas import tpu_sc as plsc

BATCH = 4096
N_FIELDS = 26
EMB_DIM = 128

NUM_CORES = 2
NUM_SUBCORES = 16
NUM_WORKERS = NUM_CORES * NUM_SUBCORES
BLK = BATCH // NUM_WORKERS


def _gather_t_body(xt_hbm, table_hbm, out_hbm, idx_v, gbufs, tbufs, *sems):
    gsems, ssems = sems[:2], sems[2:]
    wid = lax.axis_index("s") * NUM_CORES + lax.axis_index("c")
    pltpu.sync_copy(xt_hbm.at[:, pl.ds(wid * BLK, BLK)], idx_v)

    def gather(f):
        return pltpu.async_copy(
            table_hbm.at[idx_v.at[f]], gbufs.at[f % 2], gsems[f % 2]
        )

    def store(f):
        return pltpu.async_copy(tbufs.at[f % 2], out_hbm.at[f, :, wid], ssems[f % 2])

    iota = lax.iota(jnp.int32, 16)
    dts = [lax.shift_right_logical(iota + d0 * 16, 3) for d0 in range(8)]
    dss = [lax.bitwise_and(iota + d0 * 16, 7) for d0 in range(8)]

    def transpose(f):
        g = gbufs.at[f % 2]
        t = tbufs.at[f % 2]

        @pl.loop(0, BLK)
        def _row(b):
            blv = jnp.full((16,), b, jnp.int32)
            for d0 in range(8):
                vals = g[b, pl.ds(d0 * 16, 16)]
                plsc.store_scatter(t, [dts[d0], dss[d0], blv], vals)

    gathers = [None] * N_FIELDS
    stores = [None] * N_FIELDS
    gathers[0] = gather(0)
    gathers[1] = gather(1)
    for f in range(N_FIELDS):
        gathers[f].wait()
        if f >= 2:
            stores[f - 2].wait()
        transpose(f)
        stores[f] = store(f)
        if f + 2 < N_FIELDS:
            gathers[f + 2] = gather(f + 2)
    stores[N_FIELDS - 2].wait()
    stores[N_FIELDS - 1].wait()


@jax.jit
def _sc_gather_t(xt, table):
    mesh = plsc.VectorSubcoreMesh(core_axis_name="c", subcore_axis_name="s")
    k = functools.partial(
        pl.kernel,
        out_type=jax.ShapeDtypeStruct(
            (N_FIELDS, EMB_DIM // 8, NUM_WORKERS, 8, BLK), jnp.float32
        ),
        mesh=mesh,
        scratch_types=[
            pltpu.VMEM((N_FIELDS, BLK), jnp.int32),
            pltpu.VMEM((2, BLK, EMB_DIM), jnp.float32),
            pltpu.VMEM((2, EMB_DIM // 8, 8, BLK), jnp.float32),
        ]
        + [pltpu.SemaphoreType.DMA] * 4,
        compiler_params=pltpu.CompilerParams(
            use_tc_tiling_on_sc=False, needs_layout_passes=False
        ),
    )(_gather_t_body)
    return k(xt, table)


def kernel(x, table):
    raw = _sc_gather_t(x.T, table)
    return raw.transpose(2, 4, 0, 1, 3).reshape(BATCH, N_FIELDS, 1, 8, 16)
